# Initial kernel scaffold; baseline (speedup 1.0000x reference)
import jax
import jax.numpy as jnp
from jax.experimental import pallas as pl


def kernel(x, w0, w1, w2, w3, w4, w5):
    raise NotImplementedError("write your pallas kernel here")


# speedup vs baseline: 21.4641x; 21.4641x over previous
import functools

import jax
import jax.numpy as jnp
from jax import lax
from jax.experimental import pallas as pl
from jax.experimental.pallas import tpu as pltpu
from jax.experimental.pallas import tpu_sc as plsc

NC = 2
NS = 16
L = 16
NW = NC * NS

B = 262144
NLEV = 6
T_RES = (2, 2, 4, 4, 8, 8)
S_RES = (16, 24, 32, 48, 64, 80)
SIZES = tuple(T_RES[i] * S_RES[i] ** 3 for i in range(NLEV))

CHUNK = 128
PTS_PER_W = B // NW
NCHUNK = PTS_PER_W // CHUNK
GRP = CHUNK // L

RESIDENT = (True, True, False, False, False, False)


def _cell(xb, g, lev):
    rs = (T_RES[lev] - 1, S_RES[lev] - 1, S_RES[lev] - 1, S_RES[lev] - 1)
    sl = pl.ds(g * L, L)
    bli = []
    wts = []
    for c in range(4):
        xv = xb[c, sl]
        f = xv * jnp.float32(rs[c])
        b = f.astype(jnp.int32)
        b = jnp.minimum(b, rs[c] - 1)
        bli.append(b)
        wts.append(f - b.astype(jnp.float32))
    return bli, wts


def _vertex_indices(bli, lev, scale):
    X = S_RES[lev]
    SY = X
    SZ = X * X
    ST = X * X * X
    base = (bli[0] * (ST * scale) + bli[1] * scale
            + bli[2] * (SY * scale) + bli[3] * (SZ * scale))
    idxs = []
    for i in (0, 1):
        for j in (0, 1):
            for k in (0, 1):
                for l in (0, 1):
                    off = (i * ST + j + k * SY + l * SZ) * scale
                    idxs.append(base + off)
    return idxs


def _interp(e, wts):
    wt, wx, wy, wz = wts
    a = [e[m] + wt * (e[8 + m] - e[m]) for m in range(8)]
    b = [a[m] + wx * (a[4 + m] - a[m]) for m in range(4)]
    c = [b[m] + wy * (b[2 + m] - b[m]) for m in range(2)]
    return c[0] + wz * (c[1] - c[0])


def _body(xt, w0f, w1f, w2f, w3f, w4f, w5f, out, mask, tab0, tab1, xb, idxb,
          rowb, outb, maskb, sem):
    ws = (None, None, w2f, w3f, w4f, w5f)
    wid = lax.axis_index("s") * NC + lax.axis_index("c")
    base = wid * PTS_PER_W
    iota = lax.broadcasted_iota(jnp.int32, (L,), 0)
    iota12 = iota * 12

    pltpu.sync_copy(w0f, tab0)
    pltpu.sync_copy(w1f, tab1)
    tabs = (tab0, tab1)

    def process_chunk(ci, carry):
        off = base + ci * CHUNK
        pltpu.sync_copy(xt.at[:, pl.ds(off, CHUNK)], xb)

        def mask_grp(g, c2):
            sl = pl.ds(g * L, L)
            ok = None
            for c in range(4):
                xv = xb[c, sl]
                okc = (xv >= jnp.float32(0.0)) & (xv <= jnp.float32(1.0))
                ok = okc if ok is None else (ok & okc)
            maskb[sl] = jnp.where(ok, jnp.int32(1), jnp.int32(0))
            return c2

        lax.fori_loop(0, GRP, mask_grp, 0)

        for lev in range(NLEV):
            if RESIDENT[lev]:
                tab = tabs[lev]

                def grp_res(g, c2, lev=lev, tab=tab):
                    bli, wts = _cell(xb, g, lev)
                    vidx2 = _vertex_indices(bli, lev, 2)
                    rows12 = g * (12 * L) + iota12
                    for f in (0, 1):
                        e = [plsc.load_gather(tab, [vidx2[v] + f])
                             for v in range(16)]
                        o = _interp(e, wts)
                        plsc.store_scatter(outb, [rows12 + (2 * lev + f)], o)
                    return c2

                lax.fori_loop(0, GRP, grp_res, 0)
            else:
                wl = ws[lev]

                def grp_idx(g, c2, lev=lev):
                    bli, _ = _cell(xb, g, lev)
                    vidx2 = _vertex_indices(bli, lev, 2)
                    for v in range(16):
                        idxb[pl.ds((2 * v) * CHUNK + g * L, L)] = vidx2[v]
                        idxb[pl.ds((2 * v + 1) * CHUNK + g * L, L)] = (
                            vidx2[v] + 1)
                    return c2

                lax.fori_loop(0, GRP, grp_idx, 0)
                pltpu.async_copy(wl.at[idxb], rowb, sem).wait()

                def grp_int(g, c2, lev=lev):
                    _, wts = _cell(xb, g, lev)
                    rows12 = g * (12 * L) + iota12
                    for f in (0, 1):
                        e = [rowb[pl.ds((2 * v + f) * CHUNK + g * L, L)]
                             for v in range(16)]
                        o = _interp(e, wts)
                        plsc.store_scatter(outb, [rows12 + (2 * lev + f)], o)
                    return c2

                lax.fori_loop(0, GRP, grp_int, 0)

        pltpu.sync_copy(outb, out.at[pl.ds(off * 12, CHUNK * 12)])
        pltpu.sync_copy(maskb, mask.at[pl.ds(off, CHUNK)])
        return carry

    lax.fori_loop(0, NCHUNK, process_chunk, 0)


_mesh = plsc.VectorSubcoreMesh(core_axis_name="c", subcore_axis_name="s",
                               num_cores=NC, num_subcores=NS)

_hash_embed = functools.partial(
    pl.kernel,
    out_type=(
        jax.ShapeDtypeStruct((B * 12,), jnp.float32),
        jax.ShapeDtypeStruct((B,), jnp.int32),
    ),
    mesh=_mesh,
    compiler_params=pltpu.CompilerParams(
        needs_layout_passes=False, use_tc_tiling_on_sc=False),
    scratch_types=[
        pltpu.VMEM((SIZES[0] * 2,), jnp.float32),
        pltpu.VMEM((SIZES[1] * 2,), jnp.float32),
        pltpu.VMEM((4, CHUNK), jnp.float32),
        pltpu.VMEM((32 * CHUNK,), jnp.int32),
        pltpu.VMEM((32 * CHUNK,), jnp.float32),
        pltpu.VMEM((CHUNK * 12,), jnp.float32),
        pltpu.VMEM((CHUNK,), jnp.int32),
        pltpu.SemaphoreType.DMA,
    ],
)(_body)


def kernel(x, w0, w1, w2, w3, w4, w5):
    xt = x.T
    out, mask = _hash_embed(xt, w0.reshape(-1), w1.reshape(-1),
                            w2.reshape(-1), w3.reshape(-1),
                            w4.reshape(-1), w5.reshape(-1))
    return out.reshape(B, 12), mask.astype(bool)
